# Initial kernel scaffold; baseline (speedup 1.0000x reference)
import jax
import jax.numpy as jnp
from jax.experimental import pallas as pl


def kernel(mu, edge_index, W1, b1, W2, b2):
    raise NotImplementedError("write your pallas kernel here")


# speedup vs baseline: 13.5906x; 13.5906x over previous
import jax
import jax.numpy as jnp
from jax import lax
from jax.experimental import pallas as pl
from jax.experimental.pallas import tpu as pltpu
from jax.experimental.pallas import tpu_sc as plsc

NC, NS = 2, 16
NW = NC * NS
LANE = 16
CHUNK = 128
IB = 16


def _sc_scatter_add(table, row2d, col2d, zeros, npad, rt):
    mesh = plsc.VectorSubcoreMesh(
        core_axis_name="c", subcore_axis_name="s",
        num_cores=NC, num_subcores=NS)
    zrows = npad // NS

    def body(table_hbm, row_hbm, col_hbm, zero_hbm, out_hbm,
             rbuf, cbuf, vals, acc, gsem):
        c = lax.axis_index("c")
        s = lax.axis_index("s")
        wid = c * NS + s
        pltpu.sync_copy(zero_hbm.at[pl.ds(s * zrows, zrows)],
                        acc.at[pl.ds(s * zrows, zrows)])
        plsc.subcore_barrier()

        base = wid * rt

        def outer(ob, carry):
            r0 = base + ob * IB
            pltpu.sync_copy(row_hbm.at[pl.ds(r0, IB)], rbuf)
            pltpu.sync_copy(col_hbm.at[pl.ds(r0, IB)], cbuf)
            for j in range(IB):
                pltpu.async_copy(table_hbm.at[cbuf.at[j]], vals, gsem).wait()
                pltpu.sync_copy(vals, acc.at[rbuf.at[j]], add=True)
            return carry

        lax.fori_loop(0, rt // IB, outer, 0)
        plsc.subcore_barrier()
        pltpu.sync_copy(acc.at[pl.ds(s * zrows, zrows)],
                        out_hbm.at[c, pl.ds(s * zrows, zrows)])

    f = pl.kernel(
        body,
        out_type=jax.ShapeDtypeStruct((NC, npad, LANE), jnp.float32),
        mesh=mesh,
        scratch_types=[
            pltpu.VMEM((IB, CHUNK), jnp.int32),
            pltpu.VMEM((IB, CHUNK), jnp.int32),
            pltpu.VMEM((CHUNK, LANE), jnp.float32),
            pltpu.VMEM_SHARED((npad, LANE), jnp.float32),
            pltpu.SemaphoreType.DMA,
        ],
        compiler_params=pltpu.CompilerParams(use_tc_tiling_on_sc=False),
    )
    return f(table, row2d, col2d, zeros)


def _tc_normalize(partial, npad):
    blk = npad // 32

    def body(p_ref, o_ref):
        p = p_ref[...]
        ssum = p[0] + p[1]
        deg = jnp.maximum(ssum[:, 8:9], 1.0)
        o_ref[...] = ssum[:, 0:8] / deg

    return pl.pallas_call(
        body,
        grid=(32,),
        in_specs=[pl.BlockSpec((NC, blk, LANE), lambda i: (0, i, 0))],
        out_specs=pl.BlockSpec((blk, 8), lambda i: (i, 0)),
        out_shape=jax.ShapeDtypeStruct((npad, 8), jnp.float32),
    )(partial)


def _tc_mlp(xcol, w1row, b1row, w2row, b2, tot):
    bm = 2048
    h = w1row.shape[1]

    def body(x_ref, w1_ref, b1_ref, w2_ref, b2_ref, o_ref):
        x = x_ref[...]
        hid = x * w1_ref[...] + b1_ref[...]
        g = 0.5 * hid * (1.0 + lax.erf(hid * 0.7071067811865476))
        y = jnp.sum(g * w2_ref[...], axis=1, keepdims=True)
        o_ref[...] = y + b2_ref[0]

    return pl.pallas_call(
        body,
        grid=(tot // bm,),
        in_specs=[
            pl.BlockSpec((bm, 1), lambda i: (i, 0)),
            pl.BlockSpec((1, h), lambda i: (0, 0)),
            pl.BlockSpec((1, h), lambda i: (0, 0)),
            pl.BlockSpec((1, h), lambda i: (0, 0)),
            pl.BlockSpec(memory_space=pltpu.SMEM),
        ],
        out_specs=pl.BlockSpec((bm, 1), lambda i: (i, 0)),
        out_shape=jax.ShapeDtypeStruct((tot, 1), jnp.float32),
    )(xcol, w1row, b1row, w2row, b2)


def kernel(mu, edge_index, W1, b1, W2, b2):
    B, N = mu.shape
    E = edge_index.shape[1]
    H = W1.shape[0]

    npad = 100352
    rt = 784
    e_pad = NW * rt * CHUNK - E

    mu_t = mu.T
    table = jnp.concatenate(
        [mu_t, jnp.ones((N, 1), jnp.float32),
         jnp.zeros((N, LANE - B - 1), jnp.float32)], axis=1)

    row_p = jnp.concatenate(
        [edge_index[0], jnp.full((e_pad,), N, jnp.int32)]).reshape(NW * rt, CHUNK)
    col_p = jnp.concatenate(
        [edge_index[1], jnp.zeros((e_pad,), jnp.int32)]).reshape(NW * rt, CHUNK)
    zeros = jnp.zeros((npad, LANE), jnp.float32)

    partial = _sc_scatter_add(table, row_p, col_p, zeros, npad, rt)
    xn = _tc_normalize(partial, npad)

    tot = npad * B
    xcol = xn.reshape(tot, 1)
    y = _tc_mlp(xcol, W1.T, b1.reshape(1, H), W2, b2, tot)
    return y.reshape(npad, B)[:N].T


# speedup vs baseline: 24.1343x; 1.7758x over previous
import jax
import jax.numpy as jnp
from jax import lax
from jax.experimental import pallas as pl
from jax.experimental.pallas import tpu as pltpu
from jax.experimental.pallas import tpu_sc as plsc

NC, NS = 2, 16
NW = NC * NS
LANE = 16
CHUNK = 128
IB = 16


def _sc_scatter_add(table, idx3, zeros, npad, rt):
    mesh = plsc.VectorSubcoreMesh(
        core_axis_name="c", subcore_axis_name="s",
        num_cores=NC, num_subcores=NS)
    zrows = npad // NS
    nblk = rt // IB

    def body(table_hbm, idx_hbm, zero_hbm, out_hbm,
             ibuf, vals, acc, isem, gsem0, gsem1, ssem0, ssem1):
        c = lax.axis_index("c")
        s = lax.axis_index("s")
        wid = c * NS + s
        pltpu.sync_copy(zero_hbm.at[pl.ds(s * zrows, zrows)],
                        acc.at[pl.ds(s * zrows, zrows)])
        plsc.subcore_barrier()

        base = wid * rt
        gsems = [gsem0, gsem1]
        ssems = [ssem0, ssem1]

        pltpu.async_copy(idx_hbm.at[:, pl.ds(base, IB)], ibuf.at[0],
                         isem).wait()

        def outer(ob, carry):
            pb = lax.rem(ob, 2)
            cur = ibuf.at[pb]
            nxt = ibuf.at[1 - pb]

            nob = jnp.minimum(ob + 1, nblk - 1)
            nd = pltpu.async_copy(
                idx_hbm.at[:, pl.ds(base + nob * IB, IB)], nxt, isem)

            descs_g = [None, None]
            descs_s = [None, None]
            descs_g[0] = pltpu.async_copy(
                table_hbm.at[cur.at[1, 0]], vals.at[0], gsems[0])
            for j in range(IB):
                b = j & 1
                nb = (j + 1) & 1
                if j + 1 < IB:
                    if descs_s[nb] is not None:
                        descs_s[nb].wait()
                    descs_g[nb] = pltpu.async_copy(
                        table_hbm.at[cur.at[1, j + 1]], vals.at[nb], gsems[nb])
                descs_g[b].wait()
                descs_s[b] = pltpu.async_copy(
                    vals.at[b], acc.at[cur.at[0, j]], ssems[b], add=True)
            descs_s[0].wait()
            descs_s[1].wait()
            nd.wait()
            return carry

        lax.fori_loop(0, nblk, outer, 0)
        plsc.subcore_barrier()
        pltpu.sync_copy(acc.at[pl.ds(s * zrows, zrows)],
                        out_hbm.at[c, pl.ds(s * zrows, zrows)])

    f = pl.kernel(
        body,
        out_type=jax.ShapeDtypeStruct((NC, npad, LANE), jnp.float32),
        mesh=mesh,
        scratch_types=[
            pltpu.VMEM((2, 2, IB, CHUNK), jnp.int32),
            pltpu.VMEM((2, CHUNK, LANE), jnp.float32),
            pltpu.VMEM_SHARED((npad, LANE), jnp.float32),
            pltpu.SemaphoreType.DMA,
            pltpu.SemaphoreType.DMA,
            pltpu.SemaphoreType.DMA,
            pltpu.SemaphoreType.DMA,
            pltpu.SemaphoreType.DMA,
        ],
        compiler_params=pltpu.CompilerParams(use_tc_tiling_on_sc=False),
    )
    return f(table, idx3, zeros)


def _tc_normalize_t(partial, npad):
    blk = npad // 49

    def body(p_ref, o_ref):
        p = p_ref[...]
        ssum = p[0] + p[1]
        deg = jnp.maximum(ssum[:, 8:9], 1.0)
        o_ref[...] = (ssum[:, 0:8] / deg).T

    return pl.pallas_call(
        body,
        grid=(49,),
        in_specs=[pl.BlockSpec((NC, blk, LANE), lambda i: (0, i, 0))],
        out_specs=pl.BlockSpec((8, blk), lambda i: (0, i)),
        out_shape=jax.ShapeDtypeStruct((8, npad), jnp.float32),
    )(partial)


def _tc_mlp(x_bm, w1col, b1col, w2col, b2, npad):
    h = w1col.shape[0]

    def body(x_ref, w1_ref, b1_ref, w2_ref, b2_ref, o_ref):
        xv = x_ref[...][:, None, :]
        w1v = w1_ref[...][None, :, 0:1]
        b1v = b1_ref[...][None, :, 0:1]
        hid = xv * w1v + b1v
        g = 0.5 * hid * (1.0 + lax.erf(hid * 0.7071067811865476))
        y = jnp.sum(g * w2_ref[...][None, :, 0:1], axis=1)
        o_ref[...] = y + b2_ref[0]

    return pl.pallas_call(
        body,
        grid=(npad // 128,),
        in_specs=[
            pl.BlockSpec((8, 128), lambda i: (0, i)),
            pl.BlockSpec((h, 1), lambda i: (0, 0)),
            pl.BlockSpec((h, 1), lambda i: (0, 0)),
            pl.BlockSpec((h, 1), lambda i: (0, 0)),
            pl.BlockSpec(memory_space=pltpu.SMEM),
        ],
        out_specs=pl.BlockSpec((8, 128), lambda i: (0, i)),
        out_shape=jax.ShapeDtypeStruct((8, npad), jnp.float32),
    )(x_bm, w1col, b1col, w2col, b2)


def kernel(mu, edge_index, W1, b1, W2, b2):
    B, N = mu.shape
    E = edge_index.shape[1]
    H = W1.shape[0]

    npad = 100352
    rt = 784
    e_pad = NW * rt * CHUNK - E

    mu_t = mu.T
    table = jnp.concatenate(
        [mu_t, jnp.ones((N, 1), jnp.float32),
         jnp.zeros((N, LANE - B - 1), jnp.float32)], axis=1)

    pad_idx = jnp.stack([jnp.full((e_pad,), N, jnp.int32),
                         jnp.zeros((e_pad,), jnp.int32)])
    idx3 = jnp.concatenate([edge_index, pad_idx], axis=1).reshape(
        2, NW * rt, CHUNK)
    zeros = jnp.zeros((npad, LANE), jnp.float32)

    partial = _sc_scatter_add(table, idx3, zeros, npad, rt)
    x_bm = _tc_normalize_t(partial, npad)
    y = _tc_mlp(x_bm, W1, b1.reshape(H, 1), W2.reshape(H, 1), b2, npad)
    return y[:, :N]


# speedup vs baseline: 27.5273x; 1.1406x over previous
import jax
import jax.numpy as jnp
from jax import lax
from jax.experimental import pallas as pl
from jax.experimental.pallas import tpu as pltpu
from jax.experimental.pallas import tpu_sc as plsc

NC, NS = 2, 16
NW = NC * NS
LANE = 16
CHUNK = 128
IB = 16


def _sc_scatter_add(table, idx3, zeros, npad, rt):
    mesh = plsc.VectorSubcoreMesh(
        core_axis_name="c", subcore_axis_name="s",
        num_cores=NC, num_subcores=NS)
    zrows = npad // NS
    nblk = rt // IB

    def body(table_hbm, idx_hbm, zero_hbm, out_hbm,
             ibuf, vals, acc, isem, gsem0, gsem1, gsem2, gsem3,
             ssem0, ssem1, ssem2, ssem3):
        c = lax.axis_index("c")
        s = lax.axis_index("s")
        wid = c * NS + s
        pltpu.sync_copy(zero_hbm.at[pl.ds(s * zrows, zrows)],
                        acc.at[pl.ds(s * zrows, zrows)])
        plsc.subcore_barrier()

        base = wid * rt
        gsems = [gsem0, gsem1, gsem2, gsem3]
        ssems = [ssem0, ssem1, ssem2, ssem3]
        ND = 4

        pltpu.async_copy(idx_hbm.at[:, pl.ds(base, IB)], ibuf.at[0],
                         isem).wait()

        def outer(ob, carry):
            pb = lax.rem(ob, 2)
            cur = ibuf.at[pb]
            nxt = ibuf.at[1 - pb]

            nob = jnp.minimum(ob + 1, nblk - 1)
            nd = pltpu.async_copy(
                idx_hbm.at[:, pl.ds(base + nob * IB, IB)], nxt, isem)

            descs_g = [None] * ND
            descs_s = [None] * ND
            for j in range(ND - 1):
                descs_g[j] = pltpu.async_copy(
                    table_hbm.at[cur.at[1, j]], vals.at[j], gsems[j])
            for j in range(IB):
                b = j % ND
                nb = (j + ND - 1) % ND
                if j + ND - 1 < IB:
                    if descs_s[nb] is not None:
                        descs_s[nb].wait()
                    descs_g[nb] = pltpu.async_copy(
                        table_hbm.at[cur.at[1, j + ND - 1]], vals.at[nb],
                        gsems[nb])
                descs_g[b].wait()
                descs_s[b] = pltpu.async_copy(
                    vals.at[b], acc.at[cur.at[0, j]], ssems[b], add=True)
            for j in range(max(0, IB - ND), IB):
                descs_s[j % ND].wait()
            nd.wait()
            return carry

        lax.fori_loop(0, nblk, outer, 0)
        plsc.subcore_barrier()
        pltpu.sync_copy(acc.at[pl.ds(s * zrows, zrows)],
                        out_hbm.at[c, pl.ds(s * zrows, zrows)])

    f = pl.kernel(
        body,
        out_type=jax.ShapeDtypeStruct((NC, npad, LANE), jnp.float32),
        mesh=mesh,
        scratch_types=[
            pltpu.VMEM((2, 2, IB, CHUNK), jnp.int32),
            pltpu.VMEM((4, CHUNK, LANE), jnp.float32),
            pltpu.VMEM_SHARED((npad, LANE), jnp.float32),
            pltpu.SemaphoreType.DMA,
            pltpu.SemaphoreType.DMA,
            pltpu.SemaphoreType.DMA,
            pltpu.SemaphoreType.DMA,
            pltpu.SemaphoreType.DMA,
            pltpu.SemaphoreType.DMA,
            pltpu.SemaphoreType.DMA,
            pltpu.SemaphoreType.DMA,
            pltpu.SemaphoreType.DMA,
        ],
        compiler_params=pltpu.CompilerParams(use_tc_tiling_on_sc=False),
    )
    return f(table, idx3, zeros)


def _tc_normalize_t(partial, npad):
    blk = npad // 49

    def body(p_ref, o_ref):
        p = p_ref[...]
        ssum = p[0] + p[1]
        deg = jnp.maximum(ssum[:, 8:9], 1.0)
        o_ref[...] = (ssum[:, 0:8] / deg).T

    return pl.pallas_call(
        body,
        grid=(49,),
        in_specs=[pl.BlockSpec((NC, blk, LANE), lambda i: (0, i, 0))],
        out_specs=pl.BlockSpec((8, blk), lambda i: (0, i)),
        out_shape=jax.ShapeDtypeStruct((8, npad), jnp.float32),
    )(partial)


def _tc_mlp(x_bm, w1col, b1col, w2col, b2, npad):
    h = w1col.shape[0]

    def body(x_ref, w1_ref, b1_ref, w2_ref, b2_ref, o_ref):
        xv = x_ref[...][:, None, :]
        w1v = w1_ref[...][None, :, 0:1]
        b1v = b1_ref[...][None, :, 0:1]
        hid = xv * w1v + b1v
        g = 0.5 * hid * (1.0 + lax.erf(hid * 0.7071067811865476))
        y = jnp.sum(g * w2_ref[...][None, :, 0:1], axis=1)
        o_ref[...] = y + b2_ref[0]

    return pl.pallas_call(
        body,
        grid=(npad // 128,),
        in_specs=[
            pl.BlockSpec((8, 128), lambda i: (0, i)),
            pl.BlockSpec((h, 1), lambda i: (0, 0)),
            pl.BlockSpec((h, 1), lambda i: (0, 0)),
            pl.BlockSpec((h, 1), lambda i: (0, 0)),
            pl.BlockSpec(memory_space=pltpu.SMEM),
        ],
        out_specs=pl.BlockSpec((8, 128), lambda i: (0, i)),
        out_shape=jax.ShapeDtypeStruct((8, npad), jnp.float32),
    )(x_bm, w1col, b1col, w2col, b2)


def kernel(mu, edge_index, W1, b1, W2, b2):
    B, N = mu.shape
    E = edge_index.shape[1]
    H = W1.shape[0]

    npad = 100352
    rt = 784
    e_pad = NW * rt * CHUNK - E

    mu_t = mu.T
    table = jnp.concatenate(
        [mu_t, jnp.ones((N, 1), jnp.float32),
         jnp.zeros((N, LANE - B - 1), jnp.float32)], axis=1)

    pad_idx = jnp.stack([jnp.full((e_pad,), N, jnp.int32),
                         jnp.zeros((e_pad,), jnp.int32)])
    idx3 = jnp.concatenate([edge_index, pad_idx], axis=1).reshape(
        2, NW * rt, CHUNK)
    zeros = jnp.zeros((npad, LANE), jnp.float32)

    partial = _sc_scatter_add(table, idx3, zeros, npad, rt)
    x_bm = _tc_normalize_t(partial, npad)
    y = _tc_mlp(x_bm, W1, b1.reshape(H, 1), W2.reshape(H, 1), b2, npad)
    return y[:, :N]
